# Initial kernel scaffold; baseline (speedup 1.0000x reference)
import functools

import jax
import jax.numpy as jnp
from jax import lax
from jax.experimental import pallas as pl
from jax.experimental.pallas import tpu as pltpu
from jax.experimental.pallas import tpu_sc as plsc

D = 24
NBK = 1024


def kernel(product_id, popular_score, product_category,
           title_table, scores_table, category_table, buckets):
    B = product_id.shape[0]
    info = plsc.get_sparse_core_info()
    NC, NS, L = info.num_cores, info.num_subcores, info.num_lanes
    NW = NC * NS
    bpw = B // NW
    nrow = bpw // 128

    pid3 = product_id.astype(jnp.int32).reshape(NW, nrow, 128)
    cat3 = product_category.astype(jnp.int32).reshape(NW, nrow, 128)
    ps2 = popular_score.reshape(NW, bpw)
    nb = buckets.shape[0]
    bpad = jnp.concatenate(
        [buckets, jnp.full((NBK - nb,), 2.0, jnp.float32)])

    mesh = plsc.VectorSubcoreMesh(core_axis_name="c", subcore_axis_name="s")

    @functools.partial(
        pl.kernel, mesh=mesh,
        out_type=jax.ShapeDtypeStruct((B, 3 * D), jnp.float32),
        scratch_types=[
            pltpu.VMEM((nrow, 128), jnp.int32),
            pltpu.VMEM((nrow, 128), jnp.int32),
            pltpu.VMEM((nrow, 128), jnp.int32),
            pltpu.VMEM((bpw,), jnp.float32),
            pltpu.VMEM((NBK,), jnp.float32),
            pltpu.VMEM((bpw, D), jnp.float32),
            pltpu.VMEM((bpw, D), jnp.float32),
            pltpu.VMEM((bpw, D), jnp.float32),
            pltpu.SemaphoreType.DMA,
            pltpu.SemaphoreType.DMA,
            pltpu.SemaphoreType.DMA,
        ],
    )
    def sc_kernel(pid_hbm, ps_hbm, cat_hbm, ttab, stab, ctab, bkt_hbm,
                  out_hbm, pid_v, cat_v, bidx_v, ps_v, bkt_v,
                  trows, srows, crows, sem_t, sem_s, sem_c):
        wid = lax.axis_index("s") * NC + lax.axis_index("c")
        base = wid * bpw

        pltpu.sync_copy(pid_hbm.at[wid], pid_v)
        pltpu.sync_copy(cat_hbm.at[wid], cat_v)
        pltpu.sync_copy(ps_hbm.at[wid], ps_v)
        pltpu.sync_copy(bkt_hbm, bkt_v)

        copies = []
        for j in range(nrow):
            copies.append(pltpu.async_copy(
                ttab.at[pid_v.at[j]], trows.at[pl.ds(j * 128, 128)], sem_t))
            copies.append(pltpu.async_copy(
                ctab.at[cat_v.at[j]], crows.at[pl.ds(j * 128, 128)], sem_c))

        def chunk(i, carry):
            off = pl.multiple_of(i * L, L)
            v = ps_v[pl.ds(off, L)]
            pos = jnp.zeros((L,), jnp.int32)
            step = NBK // 2
            while step >= 1:
                probe = pos + (step - 1)
                bv = plsc.load_gather(bkt_v, [probe])
                pos = jnp.where(bv <= v, pos + step, pos)
                step //= 2
            row = i // 8
            col = pl.multiple_of((i % 8) * L, L)
            bidx_v[row, pl.ds(col, L)] = pos
            return carry

        lax.fori_loop(0, bpw // L, chunk, 0)

        for j in range(nrow):
            copies.append(pltpu.async_copy(
                stab.at[bidx_v.at[j]], srows.at[pl.ds(j * 128, 128)], sem_s))
        for c in copies:
            c.wait()

        pltpu.sync_copy(trows, out_hbm.at[pl.ds(base, bpw), pl.ds(0, D)])
        pltpu.sync_copy(srows, out_hbm.at[pl.ds(base, bpw), pl.ds(D, D)])
        pltpu.sync_copy(crows, out_hbm.at[pl.ds(base, bpw), pl.ds(2 * D, D)])

    return sc_kernel(pid3, ps2, cat3, title_table, scores_table,
                     category_table, bpad)


# speedup vs baseline: 10.4386x; 10.4386x over previous
import functools

import jax
import jax.numpy as jnp
from jax import lax
from jax.experimental import pallas as pl
from jax.experimental.pallas import tpu as pltpu
from jax.experimental.pallas import tpu_sc as plsc

D = 24
NBK = 1024


def kernel(product_id, popular_score, product_category,
           title_table, scores_table, category_table, buckets):
    B = product_id.shape[0]
    info = plsc.get_sparse_core_info()
    NC, NS, L = info.num_cores, info.num_subcores, info.num_lanes
    NW = NC * NS
    bpw = B // NW
    nrow = bpw // 128

    pid3 = product_id.astype(jnp.int32).reshape(NW, nrow, 128)
    cat3 = product_category.astype(jnp.int32).reshape(NW, nrow, 128)
    ps2 = popular_score.reshape(NW, bpw)
    nb = buckets.shape[0]
    bpad = jnp.concatenate(
        [buckets, jnp.full((NBK - nb,), 2.0, jnp.float32)])

    mesh = plsc.VectorSubcoreMesh(core_axis_name="c", subcore_axis_name="s")

    @functools.partial(
        pl.kernel, mesh=mesh,
        compiler_params=pltpu.CompilerParams(
            needs_layout_passes=False, use_tc_tiling_on_sc=False),
        out_type=jax.ShapeDtypeStruct((NW, bpw, 3 * D), jnp.float32),
        scratch_types=[
            pltpu.VMEM((nrow, 128), jnp.int32),
            pltpu.VMEM((nrow, 128), jnp.int32),
            pltpu.VMEM((nrow, 128), jnp.int32),
            pltpu.VMEM((bpw,), jnp.float32),
            pltpu.VMEM((NBK,), jnp.float32),
            pltpu.VMEM((bpw, D), jnp.float32),
            pltpu.VMEM((bpw, D), jnp.float32),
            pltpu.VMEM((bpw, D), jnp.float32),
            pltpu.SemaphoreType.DMA,
            pltpu.SemaphoreType.DMA,
            pltpu.SemaphoreType.DMA,
        ],
    )
    def sc_kernel(pid_hbm, ps_hbm, cat_hbm, ttab, stab, ctab, bkt_hbm,
                  out_hbm, pid_v, cat_v, bidx_v, ps_v, bkt_v,
                  trows, srows, crows, sem_t, sem_s, sem_c):
        wid = lax.axis_index("s") * NC + lax.axis_index("c")

        pltpu.sync_copy(pid_hbm.at[wid], pid_v)
        pltpu.sync_copy(cat_hbm.at[wid], cat_v)
        pltpu.sync_copy(ps_hbm.at[wid], ps_v)
        pltpu.sync_copy(bkt_hbm, bkt_v)

        copies = []
        for j in range(nrow):
            copies.append(pltpu.async_copy(
                ttab.at[pid_v.at[j]], trows.at[pl.ds(j * 128, 128)], sem_t))
            copies.append(pltpu.async_copy(
                ctab.at[cat_v.at[j]], crows.at[pl.ds(j * 128, 128)], sem_c))

        def chunk(i, carry):
            off = pl.multiple_of(i * L, L)
            v = ps_v[pl.ds(off, L)]
            pos = jnp.zeros((L,), jnp.int32)
            step = NBK // 2
            while step >= 1:
                probe = pos + (step - 1)
                bv = plsc.load_gather(bkt_v, [probe])
                pos = jnp.where(bv <= v, pos + step, pos)
                step //= 2
            row = i // 8
            col = pl.multiple_of((i % 8) * L, L)
            bidx_v[row, pl.ds(col, L)] = pos
            return carry

        lax.fori_loop(0, bpw // L, chunk, 0)

        for j in range(nrow):
            copies.append(pltpu.async_copy(
                stab.at[bidx_v.at[j]], srows.at[pl.ds(j * 128, 128)], sem_s))
        for c in copies:
            c.wait()

        pltpu.sync_copy(trows, out_hbm.at[wid, :, pl.ds(0, D)])
        pltpu.sync_copy(srows, out_hbm.at[wid, :, pl.ds(D, D)])
        pltpu.sync_copy(crows, out_hbm.at[wid, :, pl.ds(2 * D, D)])

    out = sc_kernel(pid3, ps2, cat3, title_table, scores_table,
                    category_table, bpad)
    return out.reshape(B, 3 * D)
